# baseline (speedup 1.0000x reference)
import jax
import jax.numpy as jnp
from jax import lax
from jax.experimental import pallas as pl
from jax.experimental.pallas import tpu as pltpu, tpu_sc as plsc

N = 10000
E = 320000
D = 128
H = 256
C = 40
F2 = 128

NC, NS = 2, 16
NW = NC * NS
B = 128
CHUNKS = 80
EPW = CHUNKS * B
HCH = CHUNKS // 2
EPAD = NW * EPW
NPAD = 10240
ZCH = NPAD // B // NS
NPN = NPAD // NS
RB = 2048
GRID = NPAD // RB


def _make_segsum(F, with_counts):
    mesh = plsc.VectorSubcoreMesh(core_axis_name="c", subcore_axis_name="s")
    out_type = [jax.ShapeDtypeStruct((NC, NPAD, F), jnp.float32)]
    scratch = [
        pltpu.VMEM((HCH, B), jnp.int32),
        pltpu.VMEM((HCH, B), jnp.int32),
        pltpu.VMEM((B, F), jnp.float32),
        pltpu.VMEM((B, F), jnp.float32),
        pltpu.VMEM_SHARED((NPAD, F), jnp.float32),
        pltpu.SemaphoreType.DMA,
        pltpu.SemaphoreType.DMA,
        pltpu.SemaphoreType.DMA,
        pltpu.SemaphoreType.DMA,
    ]
    if with_counts:
        out_type.append(jax.ShapeDtypeStruct((NC, NPAD), jnp.float32))
        scratch += [
            pltpu.VMEM((B,), jnp.float32),
            pltpu.VMEM((NPN,), jnp.float32),
            pltpu.VMEM_SHARED((NPAD,), jnp.float32),
        ]

    def body(table, src2d, dst2d, *refs):
        if with_counts:
            (out, cnt_out, idx_s, idx_d, rows, rows1, acc, sem, sem1,
             sem_s, sem_s1, ones, zvec, cnt_acc) = refs
        else:
            (out, idx_s, idx_d, rows, rows1, acc, sem, sem1,
             sem_s, sem_s1) = refs
        c = lax.axis_index("c")
        s = lax.axis_index("s")
        wid = c * NS + s

        def zrow(r, carry):
            for k in range(F // 16):
                rows[r, pl.ds(k * 16, 16)] = jnp.zeros((16,), jnp.float32)
            return carry
        lax.fori_loop(0, B, zrow, 0)
        for i in range(ZCH):
            base = (s * ZCH + i) * B
            pltpu.sync_copy(rows, acc.at[pl.ds(base, B)])
        if with_counts:
            for k in range(B // 16):
                ones[pl.ds(k * 16, 16)] = jnp.ones((16,), jnp.float32)
            for k in range(NPN // 16):
                zvec[pl.ds(k * 16, 16)] = jnp.zeros((16,), jnp.float32)
            pltpu.sync_copy(zvec, cnt_acc.at[pl.ds(s * NPN, NPN)])
        plsc.subcore_barrier()

        def drain_g(buf, sm, j):
            pltpu.make_async_copy(table.at[idx_s.at[j]], buf, sm).wait()

        def scat(buf, sm, j):
            h = pltpu.async_copy(buf, acc.at[idx_d.at[j]], sm, add=True)
            if with_counts:
                pltpu.sync_copy(ones, cnt_acc.at[idx_d.at[j]], add=True)
            return h

        for h in range(CHUNKS // HCH):
            base = wid * CHUNKS + h * HCH
            pltpu.sync_copy(src2d.at[pl.ds(base, HCH)], idx_s)
            pltpu.sync_copy(dst2d.at[pl.ds(base, HCH)], idx_d)

            pltpu.async_copy(table.at[idx_s.at[0]], rows, sem)
            pltpu.async_copy(table.at[idx_s.at[1]], rows1, sem1)

            def chunk2(i, carry):
                j = 2 * i
                drain_g(rows, sem, j)
                s0 = scat(rows, sem_s, j)
                drain_g(rows1, sem1, j + 1)
                s1 = scat(rows1, sem_s1, j + 1)
                s0.wait()
                pltpu.async_copy(table.at[idx_s.at[j + 2]], rows, sem)
                s1.wait()
                pltpu.async_copy(table.at[idx_s.at[j + 3]], rows1, sem1)
                return carry
            lax.fori_loop(0, (HCH - 2) // 2, chunk2, 0)

            drain_g(rows, sem, HCH - 2)
            s0 = scat(rows, sem_s, HCH - 2)
            drain_g(rows1, sem1, HCH - 1)
            s1 = scat(rows1, sem_s1, HCH - 1)
            s0.wait()
            s1.wait()
        plsc.subcore_barrier()

        for i in range(ZCH):
            base = (s * ZCH + i) * B
            pltpu.sync_copy(acc.at[pl.ds(base, B)], out.at[c, pl.ds(base, B)])
        if with_counts:
            pltpu.sync_copy(cnt_acc.at[pl.ds(s * NPN, NPN)],
                            cnt_out.at[c, pl.ds(s * NPN, NPN)])

    return pl.kernel(body, out_type=tuple(out_type), mesh=mesh,
                     scratch_types=scratch)


def _dot(a, b):
    return jnp.dot(a, b, preferred_element_type=jnp.float32)


def _tc0_body(parts, cntp, x, wl, b, wr, h_lo, h_hi, invc):
    cnt = cntp[0] + cntp[1]
    iv = 1.0 / jnp.maximum(cnt, 1.0)
    invc[...] = iv
    agg = (parts[0] + parts[1]) * iv[:, None]
    acc = _dot(agg, wl[...]) + _dot(x[...], wr[...]) + b[...]
    acc = jnp.maximum(acc, 0.0)
    h_lo[...] = acc[:, :D]
    h_hi[...] = acc[:, D:]


def _tc1_body(plo, phi, invc, h_lo, h_hi, wl_t, wl_b, b, wr_t, wr_b, wl2,
              h2, t):
    iv = invc[...]
    agg_lo = (plo[0] + plo[1]) * iv[:, None]
    agg_hi = (phi[0] + phi[1]) * iv[:, None]
    acc = (_dot(agg_lo, wl_t[...]) + _dot(agg_hi, wl_b[...])
           + _dot(h_lo[...], wr_t[...]) + _dot(h_hi[...], wr_b[...]) + b[...])
    h2v = jnp.maximum(acc, 0.0)
    h2[...] = h2v
    t[...] = _dot(h2v, wl2[...])


def _tc2_body(parts, invc, h2, wr2, b2, out):
    agg = (parts[0] + parts[1]) * invc[...][:, None]
    out[...] = agg + _dot(h2[...], wr2[...]) + b2[...]


def kernel(x, edge_index, W_l0, b_l0, W_r0, W_l1, b_l1, W_r1, W_l2, b_l2,
           W_r2):
    src = edge_index[0].astype(jnp.int32)
    dst = edge_index[1].astype(jnp.int32)
    npad_e = EPAD - E
    ar = jnp.arange(npad_e, dtype=jnp.int32)
    src2d = jnp.concatenate([src, ar % N]).reshape(EPAD // B, B)
    dst2d = jnp.concatenate([dst, N + ar % (NPAD - N)]).reshape(EPAD // B, B)

    segsum_cnt = _make_segsum(D, True)
    segsum_d = _make_segsum(D, False)
    segsum_f2 = segsum_d

    parts0, cntp = segsum_cnt(x, src2d, dst2d)
    h_lo, h_hi, invc = pl.pallas_call(
        _tc0_body,
        grid=(GRID,),
        in_specs=[
            pl.BlockSpec((2, RB, D), lambda i: (0, i, 0)),
            pl.BlockSpec((2, RB), lambda i: (0, i)),
            pl.BlockSpec((RB, D), lambda i: (i, 0)),
            pl.BlockSpec((D, H), lambda i: (0, 0)),
            pl.BlockSpec((1, H), lambda i: (0, 0)),
            pl.BlockSpec((D, H), lambda i: (0, 0)),
        ],
        out_specs=[
            pl.BlockSpec((RB, D), lambda i: (i, 0)),
            pl.BlockSpec((RB, D), lambda i: (i, 0)),
            pl.BlockSpec((RB,), lambda i: (i,)),
        ],
        out_shape=[
            jax.ShapeDtypeStruct((N, D), jnp.float32),
            jax.ShapeDtypeStruct((N, D), jnp.float32),
            jax.ShapeDtypeStruct((NPAD,), jnp.float32),
        ],
    )(parts0, cntp, x, W_l0, b_l0.reshape(1, H), W_r0)

    parts1_lo = segsum_d(h_lo, src2d, dst2d)[0]
    parts1_hi = segsum_d(h_hi, src2d, dst2d)[0]
    wl2p = jnp.concatenate([W_l2, jnp.zeros((H, F2 - C), jnp.float32)], axis=1)
    h2, t = pl.pallas_call(
        _tc1_body,
        grid=(GRID,),
        in_specs=[
            pl.BlockSpec((2, RB, D), lambda i: (0, i, 0)),
            pl.BlockSpec((2, RB, D), lambda i: (0, i, 0)),
            pl.BlockSpec((RB,), lambda i: (i,)),
            pl.BlockSpec((RB, D), lambda i: (i, 0)),
            pl.BlockSpec((RB, D), lambda i: (i, 0)),
            pl.BlockSpec((D, H), lambda i: (0, 0)),
            pl.BlockSpec((D, H), lambda i: (0, 0)),
            pl.BlockSpec((1, H), lambda i: (0, 0)),
            pl.BlockSpec((D, H), lambda i: (0, 0)),
            pl.BlockSpec((D, H), lambda i: (0, 0)),
            pl.BlockSpec((H, F2), lambda i: (0, 0)),
        ],
        out_specs=[
            pl.BlockSpec((RB, H), lambda i: (i, 0)),
            pl.BlockSpec((RB, F2), lambda i: (i, 0)),
        ],
        out_shape=[
            jax.ShapeDtypeStruct((N, H), jnp.float32),
            jax.ShapeDtypeStruct((N, F2), jnp.float32),
        ],
    )(parts1_lo, parts1_hi, invc, h_lo, h_hi,
      W_l1[:D], W_l1[D:], b_l1.reshape(1, H), W_r1[:D], W_r1[D:], wl2p)

    parts2 = segsum_f2(t, src2d, dst2d)[0]
    wr2p = jnp.concatenate([W_r2, jnp.zeros((H, F2 - C), jnp.float32)], axis=1)
    b2p = jnp.concatenate([b_l2, jnp.zeros((F2 - C,), jnp.float32)])
    out48 = pl.pallas_call(
        _tc2_body,
        grid=(GRID,),
        in_specs=[
            pl.BlockSpec((2, RB, F2), lambda i: (0, i, 0)),
            pl.BlockSpec((RB,), lambda i: (i,)),
            pl.BlockSpec((RB, H), lambda i: (i, 0)),
            pl.BlockSpec((H, F2), lambda i: (0, 0)),
            pl.BlockSpec((1, F2), lambda i: (0, 0)),
        ],
        out_specs=pl.BlockSpec((RB, F2), lambda i: (i, 0)),
        out_shape=jax.ShapeDtypeStruct((N, F2), jnp.float32),
    )(parts2, invc, h2, wr2p, b2p.reshape(1, F2))

    return out48[:, :C]


# speedup vs baseline: 1.2794x; 1.2794x over previous
import jax
import jax.numpy as jnp
from jax import lax
from jax.experimental import pallas as pl
from jax.experimental.pallas import tpu as pltpu, tpu_sc as plsc

N = 10000
E = 320000
D = 128
H = 256
C = 40
F2 = 128

NC, NS = 2, 16
NW = NC * NS
B = 128
CHUNKS = 80
EPW = CHUNKS * B
HCH = CHUNKS // 2
EPAD = NW * EPW
NPAD = 10240
ZCH = NPAD // B // NS
NPN = NPAD // NS
RB = 2048
GRID = NPAD // RB


def _make_segsum(F, with_counts):
    mesh = plsc.VectorSubcoreMesh(core_axis_name="c", subcore_axis_name="s")
    out_type = [jax.ShapeDtypeStruct((NC, NPAD, F), jnp.float32)]
    scratch = [
        pltpu.VMEM((HCH, B), jnp.int32),
        pltpu.VMEM((HCH, B), jnp.int32),
        pltpu.VMEM((B, F), jnp.float32),
        pltpu.VMEM((B, F), jnp.float32),
        pltpu.VMEM_SHARED((NPAD, F), jnp.float32),
        pltpu.SemaphoreType.DMA,
        pltpu.SemaphoreType.DMA,
    ]
    if with_counts:
        out_type.append(jax.ShapeDtypeStruct((NC, NPAD), jnp.float32))
        scratch += [
            pltpu.VMEM((B,), jnp.float32),
            pltpu.VMEM((NPN,), jnp.float32),
            pltpu.VMEM_SHARED((NPAD,), jnp.float32),
        ]

    def body(table, src2d, dst2d, *refs):
        if with_counts:
            (out, cnt_out, idx_s, idx_d, rows, rows1, acc, sem, sem1,
             ones, zvec, cnt_acc) = refs
        else:
            out, idx_s, idx_d, rows, rows1, acc, sem, sem1 = refs
        c = lax.axis_index("c")
        s = lax.axis_index("s")
        wid = c * NS + s

        def zrow(r, carry):
            for k in range(F // 16):
                rows[r, pl.ds(k * 16, 16)] = jnp.zeros((16,), jnp.float32)
            return carry
        lax.fori_loop(0, B, zrow, 0)
        for i in range(ZCH):
            base = (s * ZCH + i) * B
            pltpu.sync_copy(rows, acc.at[pl.ds(base, B)])
        if with_counts:
            for k in range(B // 16):
                ones[pl.ds(k * 16, 16)] = jnp.ones((16,), jnp.float32)
            for k in range(NPN // 16):
                zvec[pl.ds(k * 16, 16)] = jnp.zeros((16,), jnp.float32)
            pltpu.sync_copy(zvec, cnt_acc.at[pl.ds(s * NPN, NPN)])
        plsc.subcore_barrier()

        def drain_g(buf, sm, j):
            pltpu.make_async_copy(table.at[idx_s.at[j]], buf, sm).wait()

        def scat(buf, j):
            pltpu.sync_copy(buf, acc.at[idx_d.at[j]], add=True)
            if with_counts:
                pltpu.sync_copy(ones, cnt_acc.at[idx_d.at[j]], add=True)

        for h in range(CHUNKS // HCH):
            base = wid * CHUNKS + h * HCH
            pltpu.sync_copy(src2d.at[pl.ds(base, HCH)], idx_s)
            pltpu.sync_copy(dst2d.at[pl.ds(base, HCH)], idx_d)

            pltpu.async_copy(table.at[idx_s.at[0]], rows, sem)
            pltpu.async_copy(table.at[idx_s.at[1]], rows1, sem1)

            def chunk2(i, carry):
                j = 2 * i
                drain_g(rows, sem, j)
                scat(rows, j)
                pltpu.async_copy(table.at[idx_s.at[j + 2]], rows, sem)
                drain_g(rows1, sem1, j + 1)
                scat(rows1, j + 1)
                pltpu.async_copy(table.at[idx_s.at[j + 3]], rows1, sem1)
                return carry
            lax.fori_loop(0, (HCH - 2) // 2, chunk2, 0)

            drain_g(rows, sem, HCH - 2)
            scat(rows, HCH - 2)
            drain_g(rows1, sem1, HCH - 1)
            scat(rows1, HCH - 1)
        plsc.subcore_barrier()

        for i in range(ZCH):
            base = (s * ZCH + i) * B
            pltpu.sync_copy(acc.at[pl.ds(base, B)], out.at[c, pl.ds(base, B)])
        if with_counts:
            pltpu.sync_copy(cnt_acc.at[pl.ds(s * NPN, NPN)],
                            cnt_out.at[c, pl.ds(s * NPN, NPN)])

    return pl.kernel(body, out_type=tuple(out_type), mesh=mesh,
                     scratch_types=scratch)


def _dot(a, b):
    return jnp.dot(a, b, preferred_element_type=jnp.float32)


def _r0_body(x, wr, b, out):
    out[...] = _dot(x[...], wr[...]) + b[...]


def _r1_body(h_lo, h_hi, wr_t, wr_b, b, out):
    out[...] = (_dot(h_lo[...], wr_t[...]) + _dot(h_hi[...], wr_b[...])
                + b[...])


def _r2_body(h2, wr2, b2, out):
    out[...] = _dot(h2[...], wr2[...]) + b2[...]


def _tc0_body(parts, cntp, r0, wl, h_lo, h_hi, invc):
    cnt = cntp[0] + cntp[1]
    iv = 1.0 / jnp.maximum(cnt, 1.0)
    invc[...] = iv
    agg = (parts[0] + parts[1]) * iv[:, None]
    acc = jnp.maximum(_dot(agg, wl[...]) + r0[...], 0.0)
    h_lo[...] = acc[:, :D]
    h_hi[...] = acc[:, D:]


def _tc1_body(plo, phi, invc, r1, wl_t, wl_b, wl2, h2, t):
    iv = invc[...]
    agg_lo = (plo[0] + plo[1]) * iv[:, None]
    agg_hi = (phi[0] + phi[1]) * iv[:, None]
    acc = _dot(agg_lo, wl_t[...]) + _dot(agg_hi, wl_b[...]) + r1[...]
    h2v = jnp.maximum(acc, 0.0)
    h2[...] = h2v
    t[...] = _dot(h2v, wl2[...])


def _tc2_body(parts, invc, r2, out):
    agg = (parts[0] + parts[1]) * invc[...][:, None]
    out[...] = agg + r2[...]


def kernel(x, edge_index, W_l0, b_l0, W_r0, W_l1, b_l1, W_r1, W_l2, b_l2,
           W_r2):
    src = edge_index[0].astype(jnp.int32)
    dst = edge_index[1].astype(jnp.int32)
    npad_e = EPAD - E
    ar = jnp.arange(npad_e, dtype=jnp.int32)
    src2d = jnp.concatenate([src, ar % N]).reshape(EPAD // B, B)
    dst2d = jnp.concatenate([dst, N + ar % (NPAD - N)]).reshape(EPAD // B, B)

    segsum_cnt = _make_segsum(D, True)
    segsum_d = _make_segsum(D, False)
    segsum_f2 = segsum_d

    parts0, cntp = segsum_cnt(x, src2d, dst2d)
    r0 = pl.pallas_call(
        _r0_body,
        grid=(GRID,),
        in_specs=[
            pl.BlockSpec((RB, D), lambda i: (i, 0)),
            pl.BlockSpec((D, H), lambda i: (0, 0)),
            pl.BlockSpec((1, H), lambda i: (0, 0)),
        ],
        out_specs=pl.BlockSpec((RB, H), lambda i: (i, 0)),
        out_shape=jax.ShapeDtypeStruct((N, H), jnp.float32),
    )(x, W_r0, b_l0.reshape(1, H))
    h_lo, h_hi, invc = pl.pallas_call(
        _tc0_body,
        grid=(GRID,),
        in_specs=[
            pl.BlockSpec((2, RB, D), lambda i: (0, i, 0)),
            pl.BlockSpec((2, RB), lambda i: (0, i)),
            pl.BlockSpec((RB, H), lambda i: (i, 0)),
            pl.BlockSpec((D, H), lambda i: (0, 0)),
        ],
        out_specs=[
            pl.BlockSpec((RB, D), lambda i: (i, 0)),
            pl.BlockSpec((RB, D), lambda i: (i, 0)),
            pl.BlockSpec((RB,), lambda i: (i,)),
        ],
        out_shape=[
            jax.ShapeDtypeStruct((N, D), jnp.float32),
            jax.ShapeDtypeStruct((N, D), jnp.float32),
            jax.ShapeDtypeStruct((NPAD,), jnp.float32),
        ],
    )(parts0, cntp, r0, W_l0)

    parts1_lo = segsum_d(h_lo, src2d, dst2d)[0]
    parts1_hi = segsum_d(h_hi, src2d, dst2d)[0]
    r1 = pl.pallas_call(
        _r1_body,
        grid=(GRID,),
        in_specs=[
            pl.BlockSpec((RB, D), lambda i: (i, 0)),
            pl.BlockSpec((RB, D), lambda i: (i, 0)),
            pl.BlockSpec((D, H), lambda i: (0, 0)),
            pl.BlockSpec((D, H), lambda i: (0, 0)),
            pl.BlockSpec((1, H), lambda i: (0, 0)),
        ],
        out_specs=pl.BlockSpec((RB, H), lambda i: (i, 0)),
        out_shape=jax.ShapeDtypeStruct((N, H), jnp.float32),
    )(h_lo, h_hi, W_r1[:D], W_r1[D:], b_l1.reshape(1, H))
    wl2p = jnp.concatenate([W_l2, jnp.zeros((H, F2 - C), jnp.float32)], axis=1)
    h2, t = pl.pallas_call(
        _tc1_body,
        grid=(GRID,),
        in_specs=[
            pl.BlockSpec((2, RB, D), lambda i: (0, i, 0)),
            pl.BlockSpec((2, RB, D), lambda i: (0, i, 0)),
            pl.BlockSpec((RB,), lambda i: (i,)),
            pl.BlockSpec((RB, H), lambda i: (i, 0)),
            pl.BlockSpec((D, H), lambda i: (0, 0)),
            pl.BlockSpec((D, H), lambda i: (0, 0)),
            pl.BlockSpec((H, F2), lambda i: (0, 0)),
        ],
        out_specs=[
            pl.BlockSpec((RB, H), lambda i: (i, 0)),
            pl.BlockSpec((RB, F2), lambda i: (i, 0)),
        ],
        out_shape=[
            jax.ShapeDtypeStruct((N, H), jnp.float32),
            jax.ShapeDtypeStruct((N, F2), jnp.float32),
        ],
    )(parts1_lo, parts1_hi, invc, r1, W_l1[:D], W_l1[D:], wl2p)

    parts2 = segsum_f2(t, src2d, dst2d)[0]
    wr2p = jnp.concatenate([W_r2, jnp.zeros((H, F2 - C), jnp.float32)], axis=1)
    b2p = jnp.concatenate([b_l2, jnp.zeros((F2 - C,), jnp.float32)])
    r2 = pl.pallas_call(
        _r2_body,
        grid=(GRID,),
        in_specs=[
            pl.BlockSpec((RB, H), lambda i: (i, 0)),
            pl.BlockSpec((H, F2), lambda i: (0, 0)),
            pl.BlockSpec((1, F2), lambda i: (0, 0)),
        ],
        out_specs=pl.BlockSpec((RB, F2), lambda i: (i, 0)),
        out_shape=jax.ShapeDtypeStruct((N, F2), jnp.float32),
    )(h2, wr2p, b2p.reshape(1, F2))
    out48 = pl.pallas_call(
        _tc2_body,
        grid=(GRID,),
        in_specs=[
            pl.BlockSpec((2, RB, F2), lambda i: (0, i, 0)),
            pl.BlockSpec((RB,), lambda i: (i,)),
            pl.BlockSpec((RB, F2), lambda i: (i, 0)),
        ],
        out_specs=pl.BlockSpec((RB, F2), lambda i: (i, 0)),
        out_shape=jax.ShapeDtypeStruct((N, F2), jnp.float32),
    )(parts2, invc, r2)

    return out48[:, :C]


# speedup vs baseline: 1.2807x; 1.0010x over previous
import jax
import jax.numpy as jnp
from jax import lax
from jax.experimental import pallas as pl
from jax.experimental.pallas import tpu as pltpu, tpu_sc as plsc

N = 10000
E = 320000
D = 128
H = 256
C = 40
F2 = 128

NC, NS = 2, 16
NW = NC * NS
B = 128
NBUF = 2
CHUNKS = 80
EPW = CHUNKS * B
HCH = CHUNKS // 2
EPAD = NW * EPW
NPAD = 10240
ZCH = NPAD // B // NS
NPN = NPAD // NS
RB = 2048
GRID = NPAD // RB


def _make_segsum(F, with_counts):
    mesh = plsc.VectorSubcoreMesh(core_axis_name="c", subcore_axis_name="s")
    out_type = [jax.ShapeDtypeStruct((NC, NPAD, F), jnp.float32)]
    scratch = (
        [pltpu.VMEM((HCH, B), jnp.int32),
         pltpu.VMEM((HCH, B), jnp.int32)]
        + [pltpu.VMEM((B, F), jnp.float32) for _ in range(NBUF)]
        + [pltpu.VMEM_SHARED((NPAD, F), jnp.float32)]
        + [pltpu.SemaphoreType.DMA for _ in range(NBUF)]
    )
    if with_counts:
        out_type.append(jax.ShapeDtypeStruct((NC, NPAD), jnp.float32))
        scratch += [
            pltpu.VMEM((B,), jnp.float32),
            pltpu.VMEM((NPN,), jnp.float32),
            pltpu.VMEM_SHARED((NPAD,), jnp.float32),
        ]

    def body(table, src2d, dst2d, *refs):
        if with_counts:
            out, cnt_out = refs[0], refs[1]
            refs = refs[2:]
            ones, zvec, cnt_acc = refs[3 + 2 * NBUF:]
        else:
            out = refs[0]
            refs = refs[1:]
        idx_s, idx_d = refs[0], refs[1]
        bufs = refs[2:2 + NBUF]
        acc = refs[2 + NBUF]
        sems = refs[3 + NBUF:3 + 2 * NBUF]
        rows = bufs[0]
        c = lax.axis_index("c")
        s = lax.axis_index("s")
        wid = c * NS + s

        def zrow(r, carry):
            for k in range(F // 16):
                rows[r, pl.ds(k * 16, 16)] = jnp.zeros((16,), jnp.float32)
            return carry
        lax.fori_loop(0, B, zrow, 0)
        for i in range(ZCH):
            base = (s * ZCH + i) * B
            pltpu.sync_copy(rows, acc.at[pl.ds(base, B)])
        if with_counts:
            for k in range(B // 16):
                ones[pl.ds(k * 16, 16)] = jnp.ones((16,), jnp.float32)
            for k in range(NPN // 16):
                zvec[pl.ds(k * 16, 16)] = jnp.zeros((16,), jnp.float32)
            pltpu.sync_copy(zvec, cnt_acc.at[pl.ds(s * NPN, NPN)])
        plsc.subcore_barrier()

        def drain_g(buf, sm, j):
            pltpu.make_async_copy(table.at[idx_s.at[j]], buf, sm).wait()

        def scat(buf, j):
            pltpu.sync_copy(buf, acc.at[idx_d.at[j]], add=True)
            if with_counts:
                pltpu.sync_copy(ones, cnt_acc.at[idx_d.at[j]], add=True)

        for h in range(CHUNKS // HCH):
            base = wid * CHUNKS + h * HCH
            pltpu.sync_copy(src2d.at[pl.ds(base, HCH)], idx_s)
            pltpu.sync_copy(dst2d.at[pl.ds(base, HCH)], idx_d)

            for b in range(NBUF):
                pltpu.async_copy(table.at[idx_s.at[b]], bufs[b], sems[b])

            def chunkn(i, carry):
                j = NBUF * i
                for b in range(NBUF):
                    drain_g(bufs[b], sems[b], j + b)
                    scat(bufs[b], j + b)
                    pltpu.async_copy(table.at[idx_s.at[j + NBUF + b]],
                                     bufs[b], sems[b])
                return carry
            lax.fori_loop(0, (HCH - NBUF) // NBUF, chunkn, 0)

            for b in range(NBUF):
                drain_g(bufs[b], sems[b], HCH - NBUF + b)
                scat(bufs[b], HCH - NBUF + b)
        plsc.subcore_barrier()

        for i in range(ZCH):
            base = (s * ZCH + i) * B
            pltpu.sync_copy(acc.at[pl.ds(base, B)], out.at[c, pl.ds(base, B)])
        if with_counts:
            pltpu.sync_copy(cnt_acc.at[pl.ds(s * NPN, NPN)],
                            cnt_out.at[c, pl.ds(s * NPN, NPN)])

    return pl.kernel(body, out_type=tuple(out_type), mesh=mesh,
                     scratch_types=scratch)


def _dot(a, b):
    return jnp.dot(a, b, preferred_element_type=jnp.float32)


def _r0_body(x, wr, b, out):
    out[...] = _dot(x[...], wr[...]) + b[...]


def _r1_body(h_lo, h_hi, wr_t, wr_b, b, out):
    out[...] = (_dot(h_lo[...], wr_t[...]) + _dot(h_hi[...], wr_b[...])
                + b[...])


def _r2_body(h2, wr2, b2, out):
    out[...] = _dot(h2[...], wr2[...]) + b2[...]


def _tc0_body(parts, cntp, r0, wl, h_lo, h_hi, invc):
    cnt = cntp[0] + cntp[1]
    iv = 1.0 / jnp.maximum(cnt, 1.0)
    invc[...] = iv
    agg = (parts[0] + parts[1]) * iv[:, None]
    acc = jnp.maximum(_dot(agg, wl[...]) + r0[...], 0.0)
    h_lo[...] = acc[:, :D]
    h_hi[...] = acc[:, D:]


def _tc1_body(plo, phi, invc, r1, wl_t, wl_b, wl2, h2, t):
    iv = invc[...]
    agg_lo = (plo[0] + plo[1]) * iv[:, None]
    agg_hi = (phi[0] + phi[1]) * iv[:, None]
    acc = _dot(agg_lo, wl_t[...]) + _dot(agg_hi, wl_b[...]) + r1[...]
    h2v = jnp.maximum(acc, 0.0)
    h2[...] = h2v
    t[...] = _dot(h2v, wl2[...])


def _tc2_body(parts, invc, r2, out):
    agg = (parts[0] + parts[1]) * invc[...][:, None]
    out[...] = agg + r2[...]


def kernel(x, edge_index, W_l0, b_l0, W_r0, W_l1, b_l1, W_r1, W_l2, b_l2,
           W_r2):
    src = edge_index[0].astype(jnp.int32)
    dst = edge_index[1].astype(jnp.int32)
    npad_e = EPAD - E
    ar = jnp.arange(npad_e, dtype=jnp.int32)
    src2d = jnp.concatenate([src, ar % N]).reshape(EPAD // B, B)
    dst2d = jnp.concatenate([dst, N + ar % (NPAD - N)]).reshape(EPAD // B, B)

    segsum_cnt = _make_segsum(D, True)
    segsum_d = _make_segsum(D, False)
    segsum_f2 = segsum_d

    parts0, cntp = segsum_cnt(x, src2d, dst2d)
    r0 = pl.pallas_call(
        _r0_body,
        grid=(GRID,),
        in_specs=[
            pl.BlockSpec((RB, D), lambda i: (i, 0)),
            pl.BlockSpec((D, H), lambda i: (0, 0)),
            pl.BlockSpec((1, H), lambda i: (0, 0)),
        ],
        out_specs=pl.BlockSpec((RB, H), lambda i: (i, 0)),
        out_shape=jax.ShapeDtypeStruct((N, H), jnp.float32),
    )(x, W_r0, b_l0.reshape(1, H))
    h_lo, h_hi, invc = pl.pallas_call(
        _tc0_body,
        grid=(GRID,),
        in_specs=[
            pl.BlockSpec((2, RB, D), lambda i: (0, i, 0)),
            pl.BlockSpec((2, RB), lambda i: (0, i)),
            pl.BlockSpec((RB, H), lambda i: (i, 0)),
            pl.BlockSpec((D, H), lambda i: (0, 0)),
        ],
        out_specs=[
            pl.BlockSpec((RB, D), lambda i: (i, 0)),
            pl.BlockSpec((RB, D), lambda i: (i, 0)),
            pl.BlockSpec((RB,), lambda i: (i,)),
        ],
        out_shape=[
            jax.ShapeDtypeStruct((N, D), jnp.float32),
            jax.ShapeDtypeStruct((N, D), jnp.float32),
            jax.ShapeDtypeStruct((NPAD,), jnp.float32),
        ],
    )(parts0, cntp, r0, W_l0)

    parts1_lo = segsum_d(h_lo, src2d, dst2d)[0]
    parts1_hi = segsum_d(h_hi, src2d, dst2d)[0]
    r1 = pl.pallas_call(
        _r1_body,
        grid=(GRID,),
        in_specs=[
            pl.BlockSpec((RB, D), lambda i: (i, 0)),
            pl.BlockSpec((RB, D), lambda i: (i, 0)),
            pl.BlockSpec((D, H), lambda i: (0, 0)),
            pl.BlockSpec((D, H), lambda i: (0, 0)),
            pl.BlockSpec((1, H), lambda i: (0, 0)),
        ],
        out_specs=pl.BlockSpec((RB, H), lambda i: (i, 0)),
        out_shape=jax.ShapeDtypeStruct((N, H), jnp.float32),
    )(h_lo, h_hi, W_r1[:D], W_r1[D:], b_l1.reshape(1, H))
    wl2p = jnp.concatenate([W_l2, jnp.zeros((H, F2 - C), jnp.float32)], axis=1)
    h2, t = pl.pallas_call(
        _tc1_body,
        grid=(GRID,),
        in_specs=[
            pl.BlockSpec((2, RB, D), lambda i: (0, i, 0)),
            pl.BlockSpec((2, RB, D), lambda i: (0, i, 0)),
            pl.BlockSpec((RB,), lambda i: (i,)),
            pl.BlockSpec((RB, H), lambda i: (i, 0)),
            pl.BlockSpec((D, H), lambda i: (0, 0)),
            pl.BlockSpec((D, H), lambda i: (0, 0)),
            pl.BlockSpec((H, F2), lambda i: (0, 0)),
        ],
        out_specs=[
            pl.BlockSpec((RB, H), lambda i: (i, 0)),
            pl.BlockSpec((RB, F2), lambda i: (i, 0)),
        ],
        out_shape=[
            jax.ShapeDtypeStruct((N, H), jnp.float32),
            jax.ShapeDtypeStruct((N, F2), jnp.float32),
        ],
    )(parts1_lo, parts1_hi, invc, r1, W_l1[:D], W_l1[D:], wl2p)

    parts2 = segsum_f2(t, src2d, dst2d)[0]
    wr2p = jnp.concatenate([W_r2, jnp.zeros((H, F2 - C), jnp.float32)], axis=1)
    b2p = jnp.concatenate([b_l2, jnp.zeros((F2 - C,), jnp.float32)])
    r2 = pl.pallas_call(
        _r2_body,
        grid=(GRID,),
        in_specs=[
            pl.BlockSpec((RB, H), lambda i: (i, 0)),
            pl.BlockSpec((H, F2), lambda i: (0, 0)),
            pl.BlockSpec((1, F2), lambda i: (0, 0)),
        ],
        out_specs=pl.BlockSpec((RB, F2), lambda i: (i, 0)),
        out_shape=jax.ShapeDtypeStruct((N, F2), jnp.float32),
    )(h2, wr2p, b2p.reshape(1, F2))
    out48 = pl.pallas_call(
        _tc2_body,
        grid=(GRID,),
        in_specs=[
            pl.BlockSpec((2, RB, F2), lambda i: (0, i, 0)),
            pl.BlockSpec((RB,), lambda i: (i,)),
            pl.BlockSpec((RB, F2), lambda i: (i, 0)),
        ],
        out_specs=pl.BlockSpec((RB, F2), lambda i: (i, 0)),
        out_shape=jax.ShapeDtypeStruct((N, F2), jnp.float32),
    )(parts2, invc, r2)

    return out48[:, :C]


# speedup vs baseline: 1.3196x; 1.0304x over previous
import jax
import jax.numpy as jnp
from jax import lax
from jax.experimental import pallas as pl
from jax.experimental.pallas import tpu as pltpu, tpu_sc as plsc

N = 10000
E = 320000
D = 128
H = 256
C = 40
F2 = 128

NC, NS = 2, 16
NW = NC * NS
B = 64
NBUF = 4
CHUNKS = 160
EPW = CHUNKS * B
HCH = CHUNKS // 4
EPAD = NW * EPW
NPAD = 10240
ZCH = NPAD // B // NS
NPN = NPAD // NS
RB = 2048
GRID = NPAD // RB


def _make_segsum(F, with_counts):
    mesh = plsc.VectorSubcoreMesh(core_axis_name="c", subcore_axis_name="s")
    out_type = [jax.ShapeDtypeStruct((NC, NPAD, F), jnp.float32)]
    scratch = (
        [pltpu.VMEM((HCH, B), jnp.int32),
         pltpu.VMEM((HCH, B), jnp.int32)]
        + [pltpu.VMEM((B, F), jnp.float32) for _ in range(NBUF)]
        + [pltpu.VMEM_SHARED((NPAD, F), jnp.float32)]
        + [pltpu.SemaphoreType.DMA for _ in range(NBUF)]
    )
    if with_counts:
        out_type.append(jax.ShapeDtypeStruct((NC, NPAD), jnp.float32))
        scratch += [
            pltpu.VMEM((B,), jnp.float32),
            pltpu.VMEM((NPN,), jnp.float32),
            pltpu.VMEM_SHARED((NPAD,), jnp.float32),
        ]

    def body(table, src2d, dst2d, *refs):
        if with_counts:
            out, cnt_out = refs[0], refs[1]
            refs = refs[2:]
            ones, zvec, cnt_acc = refs[3 + 2 * NBUF:]
        else:
            out = refs[0]
            refs = refs[1:]
        idx_s, idx_d = refs[0], refs[1]
        bufs = refs[2:2 + NBUF]
        acc = refs[2 + NBUF]
        sems = refs[3 + NBUF:3 + 2 * NBUF]
        rows = bufs[0]
        c = lax.axis_index("c")
        s = lax.axis_index("s")
        wid = c * NS + s

        def zrow(r, carry):
            for k in range(F // 16):
                rows[r, pl.ds(k * 16, 16)] = jnp.zeros((16,), jnp.float32)
            return carry
        lax.fori_loop(0, B, zrow, 0)
        for i in range(ZCH):
            base = (s * ZCH + i) * B
            pltpu.sync_copy(rows, acc.at[pl.ds(base, B)])
        if with_counts:
            for k in range(B // 16):
                ones[pl.ds(k * 16, 16)] = jnp.ones((16,), jnp.float32)
            for k in range(NPN // 16):
                zvec[pl.ds(k * 16, 16)] = jnp.zeros((16,), jnp.float32)
            pltpu.sync_copy(zvec, cnt_acc.at[pl.ds(s * NPN, NPN)])
        plsc.subcore_barrier()

        def drain_g(buf, sm, j):
            pltpu.make_async_copy(table.at[idx_s.at[j]], buf, sm).wait()

        def scat(buf, j):
            pltpu.sync_copy(buf, acc.at[idx_d.at[j]], add=True)
            if with_counts:
                pltpu.sync_copy(ones, cnt_acc.at[idx_d.at[j]], add=True)

        for h in range(CHUNKS // HCH):
            base = wid * CHUNKS + h * HCH
            pltpu.sync_copy(src2d.at[pl.ds(base, HCH)], idx_s)
            pltpu.sync_copy(dst2d.at[pl.ds(base, HCH)], idx_d)

            for b in range(NBUF):
                pltpu.async_copy(table.at[idx_s.at[b]], bufs[b], sems[b])

            def chunkn(i, carry):
                j = NBUF * i
                for b in range(NBUF):
                    drain_g(bufs[b], sems[b], j + b)
                    scat(bufs[b], j + b)
                    pltpu.async_copy(table.at[idx_s.at[j + NBUF + b]],
                                     bufs[b], sems[b])
                return carry
            lax.fori_loop(0, (HCH - NBUF) // NBUF, chunkn, 0)

            for b in range(NBUF):
                drain_g(bufs[b], sems[b], HCH - NBUF + b)
                scat(bufs[b], HCH - NBUF + b)
        plsc.subcore_barrier()

        for i in range(ZCH):
            base = (s * ZCH + i) * B
            pltpu.sync_copy(acc.at[pl.ds(base, B)], out.at[c, pl.ds(base, B)])
        if with_counts:
            pltpu.sync_copy(cnt_acc.at[pl.ds(s * NPN, NPN)],
                            cnt_out.at[c, pl.ds(s * NPN, NPN)])

    return pl.kernel(body, out_type=tuple(out_type), mesh=mesh,
                     scratch_types=scratch)


def _dot(a, b):
    return jnp.dot(a, b, preferred_element_type=jnp.float32)


def _r0_body(x, wr, b, out):
    out[...] = _dot(x[...], wr[...]) + b[...]


def _r1_body(h_lo, h_hi, wr_t, wr_b, b, out):
    out[...] = (_dot(h_lo[...], wr_t[...]) + _dot(h_hi[...], wr_b[...])
                + b[...])


def _r2_body(h2, wr2, b2, out):
    out[...] = _dot(h2[...], wr2[...]) + b2[...]


def _tc0_body(parts, cntp, r0, wl, h_lo, h_hi, invc):
    cnt = cntp[0] + cntp[1]
    iv = 1.0 / jnp.maximum(cnt, 1.0)
    invc[...] = iv
    agg = (parts[0] + parts[1]) * iv[:, None]
    acc = jnp.maximum(_dot(agg, wl[...]) + r0[...], 0.0)
    h_lo[...] = acc[:, :D]
    h_hi[...] = acc[:, D:]


def _tc1_body(plo, phi, invc, r1, wl_t, wl_b, wl2, h2, t):
    iv = invc[...]
    agg_lo = (plo[0] + plo[1]) * iv[:, None]
    agg_hi = (phi[0] + phi[1]) * iv[:, None]
    acc = _dot(agg_lo, wl_t[...]) + _dot(agg_hi, wl_b[...]) + r1[...]
    h2v = jnp.maximum(acc, 0.0)
    h2[...] = h2v
    t[...] = _dot(h2v, wl2[...])


def _tc2_body(parts, invc, r2, out):
    agg = (parts[0] + parts[1]) * invc[...][:, None]
    out[...] = agg + r2[...]


def kernel(x, edge_index, W_l0, b_l0, W_r0, W_l1, b_l1, W_r1, W_l2, b_l2,
           W_r2):
    src = edge_index[0].astype(jnp.int32)
    dst = edge_index[1].astype(jnp.int32)
    npad_e = EPAD - E
    ar = jnp.arange(npad_e, dtype=jnp.int32)
    src2d = jnp.concatenate([src, ar % N]).reshape(EPAD // B, B)
    dst2d = jnp.concatenate([dst, N + ar % (NPAD - N)]).reshape(EPAD // B, B)

    segsum_cnt = _make_segsum(D, True)
    segsum_d = _make_segsum(D, False)
    segsum_f2 = segsum_d

    parts0, cntp = segsum_cnt(x, src2d, dst2d)
    r0 = pl.pallas_call(
        _r0_body,
        grid=(GRID,),
        in_specs=[
            pl.BlockSpec((RB, D), lambda i: (i, 0)),
            pl.BlockSpec((D, H), lambda i: (0, 0)),
            pl.BlockSpec((1, H), lambda i: (0, 0)),
        ],
        out_specs=pl.BlockSpec((RB, H), lambda i: (i, 0)),
        out_shape=jax.ShapeDtypeStruct((N, H), jnp.float32),
    )(x, W_r0, b_l0.reshape(1, H))
    h_lo, h_hi, invc = pl.pallas_call(
        _tc0_body,
        grid=(GRID,),
        in_specs=[
            pl.BlockSpec((2, RB, D), lambda i: (0, i, 0)),
            pl.BlockSpec((2, RB), lambda i: (0, i)),
            pl.BlockSpec((RB, H), lambda i: (i, 0)),
            pl.BlockSpec((D, H), lambda i: (0, 0)),
        ],
        out_specs=[
            pl.BlockSpec((RB, D), lambda i: (i, 0)),
            pl.BlockSpec((RB, D), lambda i: (i, 0)),
            pl.BlockSpec((RB,), lambda i: (i,)),
        ],
        out_shape=[
            jax.ShapeDtypeStruct((N, D), jnp.float32),
            jax.ShapeDtypeStruct((N, D), jnp.float32),
            jax.ShapeDtypeStruct((NPAD,), jnp.float32),
        ],
    )(parts0, cntp, r0, W_l0)

    parts1_lo = segsum_d(h_lo, src2d, dst2d)[0]
    parts1_hi = segsum_d(h_hi, src2d, dst2d)[0]
    r1 = pl.pallas_call(
        _r1_body,
        grid=(GRID,),
        in_specs=[
            pl.BlockSpec((RB, D), lambda i: (i, 0)),
            pl.BlockSpec((RB, D), lambda i: (i, 0)),
            pl.BlockSpec((D, H), lambda i: (0, 0)),
            pl.BlockSpec((D, H), lambda i: (0, 0)),
            pl.BlockSpec((1, H), lambda i: (0, 0)),
        ],
        out_specs=pl.BlockSpec((RB, H), lambda i: (i, 0)),
        out_shape=jax.ShapeDtypeStruct((N, H), jnp.float32),
    )(h_lo, h_hi, W_r1[:D], W_r1[D:], b_l1.reshape(1, H))
    wl2p = jnp.concatenate([W_l2, jnp.zeros((H, F2 - C), jnp.float32)], axis=1)
    h2, t = pl.pallas_call(
        _tc1_body,
        grid=(GRID,),
        in_specs=[
            pl.BlockSpec((2, RB, D), lambda i: (0, i, 0)),
            pl.BlockSpec((2, RB, D), lambda i: (0, i, 0)),
            pl.BlockSpec((RB,), lambda i: (i,)),
            pl.BlockSpec((RB, H), lambda i: (i, 0)),
            pl.BlockSpec((D, H), lambda i: (0, 0)),
            pl.BlockSpec((D, H), lambda i: (0, 0)),
            pl.BlockSpec((H, F2), lambda i: (0, 0)),
        ],
        out_specs=[
            pl.BlockSpec((RB, H), lambda i: (i, 0)),
            pl.BlockSpec((RB, F2), lambda i: (i, 0)),
        ],
        out_shape=[
            jax.ShapeDtypeStruct((N, H), jnp.float32),
            jax.ShapeDtypeStruct((N, F2), jnp.float32),
        ],
    )(parts1_lo, parts1_hi, invc, r1, W_l1[:D], W_l1[D:], wl2p)

    parts2 = segsum_f2(t, src2d, dst2d)[0]
    wr2p = jnp.concatenate([W_r2, jnp.zeros((H, F2 - C), jnp.float32)], axis=1)
    b2p = jnp.concatenate([b_l2, jnp.zeros((F2 - C,), jnp.float32)])
    r2 = pl.pallas_call(
        _r2_body,
        grid=(GRID,),
        in_specs=[
            pl.BlockSpec((RB, H), lambda i: (i, 0)),
            pl.BlockSpec((H, F2), lambda i: (0, 0)),
            pl.BlockSpec((1, F2), lambda i: (0, 0)),
        ],
        out_specs=pl.BlockSpec((RB, F2), lambda i: (i, 0)),
        out_shape=jax.ShapeDtypeStruct((N, F2), jnp.float32),
    )(h2, wr2p, b2p.reshape(1, F2))
    out48 = pl.pallas_call(
        _tc2_body,
        grid=(GRID,),
        in_specs=[
            pl.BlockSpec((2, RB, F2), lambda i: (0, i, 0)),
            pl.BlockSpec((RB,), lambda i: (i,)),
            pl.BlockSpec((RB, F2), lambda i: (i, 0)),
        ],
        out_specs=pl.BlockSpec((RB, F2), lambda i: (i, 0)),
        out_shape=jax.ShapeDtypeStruct((N, F2), jnp.float32),
    )(parts2, invc, r2)

    return out48[:, :C]


# speedup vs baseline: 1.3250x; 1.0041x over previous
import jax
import jax.numpy as jnp
from jax import lax
from jax.experimental import pallas as pl
from jax.experimental.pallas import tpu as pltpu, tpu_sc as plsc

N = 10000
E = 320000
D = 128
H = 256
C = 40
F2 = 128

NC, NS = 2, 16
NW = NC * NS
B = 64
NBUF = 4
CHUNKS = 160
EPW = CHUNKS * B
HCH = CHUNKS // 4
EPAD = NW * EPW
NPAD = 10240
ZCH = NPAD // B // NS
NPN = NPAD // NS
RB = 2048
GRID = NPAD // RB


def _make_segsum(F, with_counts):
    mesh = plsc.VectorSubcoreMesh(core_axis_name="c", subcore_axis_name="s")
    out_type = [jax.ShapeDtypeStruct((NC, NPAD, F), jnp.float32)]
    scratch = (
        [pltpu.VMEM((HCH, B), jnp.int32),
         pltpu.VMEM((HCH, B), jnp.int32)]
        + [pltpu.VMEM((B, F), jnp.float32) for _ in range(NBUF)]
        + [pltpu.VMEM_SHARED((NPAD, F), jnp.float32)]
        + [pltpu.SemaphoreType.DMA for _ in range(NBUF)]
    )
    if with_counts:
        out_type.append(jax.ShapeDtypeStruct((NC, NPAD), jnp.float32))
        scratch += [
            pltpu.VMEM((B,), jnp.float32),
            pltpu.VMEM((NPN,), jnp.float32),
            pltpu.VMEM_SHARED((NPAD,), jnp.float32),
        ]

    def body(table, src2d, dst2d, *refs):
        if with_counts:
            out, cnt_out = refs[0], refs[1]
            refs = refs[2:]
            ones, zvec, cnt_acc = refs[3 + 2 * NBUF:]
        else:
            out = refs[0]
            refs = refs[1:]
        idx_s, idx_d = refs[0], refs[1]
        bufs = refs[2:2 + NBUF]
        acc = refs[2 + NBUF]
        sems = refs[3 + NBUF:3 + 2 * NBUF]
        rows = bufs[0]
        c = lax.axis_index("c")
        s = lax.axis_index("s")
        wid = c * NS + s

        def zrow(r, carry):
            for k in range(F // 16):
                rows[r, pl.ds(k * 16, 16)] = jnp.zeros((16,), jnp.float32)
            return carry
        lax.fori_loop(0, B, zrow, 0)
        for i in range(ZCH):
            base = (s * ZCH + i) * B
            pltpu.sync_copy(rows, acc.at[pl.ds(base, B)])
        if with_counts:
            for k in range(B // 16):
                ones[pl.ds(k * 16, 16)] = jnp.ones((16,), jnp.float32)
            for k in range(NPN // 16):
                zvec[pl.ds(k * 16, 16)] = jnp.zeros((16,), jnp.float32)
            pltpu.sync_copy(zvec, cnt_acc.at[pl.ds(s * NPN, NPN)])
        plsc.subcore_barrier()

        def drain_g(buf, sm, j):
            pltpu.make_async_copy(table.at[idx_s.at[j]], buf, sm).wait()

        def scat(buf, j):
            pltpu.sync_copy(buf, acc.at[idx_d.at[j]], add=True)
            if with_counts:
                pltpu.sync_copy(ones, cnt_acc.at[idx_d.at[j]], add=True)

        for h in range(CHUNKS // HCH):
            base = wid * CHUNKS + h * HCH
            pltpu.sync_copy(src2d.at[pl.ds(base, HCH)], idx_s)
            pltpu.sync_copy(dst2d.at[pl.ds(base, HCH)], idx_d)

            for b in range(NBUF):
                pltpu.async_copy(table.at[idx_s.at[b]], bufs[b], sems[b])

            def chunkn(i, carry):
                j = NBUF * i
                for b in range(NBUF):
                    drain_g(bufs[b], sems[b], j + b)
                    scat(bufs[b], j + b)
                    pltpu.async_copy(table.at[idx_s.at[j + NBUF + b]],
                                     bufs[b], sems[b])
                return carry
            lax.fori_loop(0, (HCH - NBUF) // NBUF, chunkn, 0)

            for b in range(NBUF):
                drain_g(bufs[b], sems[b], HCH - NBUF + b)
                scat(bufs[b], HCH - NBUF + b)
        plsc.subcore_barrier()

        for i in range(ZCH):
            base = (s * ZCH + i) * B
            pltpu.sync_copy(acc.at[pl.ds(base, B)], out.at[c, pl.ds(base, B)])
        if with_counts:
            pltpu.sync_copy(cnt_acc.at[pl.ds(s * NPN, NPN)],
                            cnt_out.at[c, pl.ds(s * NPN, NPN)])

    return pl.kernel(body, out_type=tuple(out_type), mesh=mesh,
                     scratch_types=scratch)


def _dot(a, b):
    return jnp.dot(a, b, preferred_element_type=jnp.float32)


def _tc0_body(parts, cntp, x, wr, b0, wl, h_lo, h_hi, invc):
    cnt = cntp[0] + cntp[1]
    iv = 1.0 / jnp.maximum(cnt, 1.0)
    invc[...] = iv
    r0 = _dot(x[...], wr[...]) + b0[...]
    agg = (parts[0] + parts[1]) * iv[:, None]
    acc = jnp.maximum(_dot(agg, wl[...]) + r0, 0.0)
    h_lo[...] = acc[:, :D]
    h_hi[...] = acc[:, D:]


def _tc1_body(plo, phi, invc, h_lo, h_hi, wr_t, wr_b, b1, wl_t, wl_b, wl2,
              h2, t):
    iv = invc[...]
    r1 = _dot(h_lo[...], wr_t[...]) + _dot(h_hi[...], wr_b[...]) + b1[...]
    agg_lo = (plo[0] + plo[1]) * iv[:, None]
    agg_hi = (phi[0] + phi[1]) * iv[:, None]
    acc = _dot(agg_lo, wl_t[...]) + _dot(agg_hi, wl_b[...]) + r1
    h2v = jnp.maximum(acc, 0.0)
    h2[...] = h2v
    t[...] = _dot(h2v, wl2[...])


def _tc2_body(parts, invc, h2, wr2, b2, out):
    r2 = _dot(h2[...], wr2[...]) + b2[...]
    agg = (parts[0] + parts[1]) * invc[...][:, None]
    out[...] = agg + r2


def kernel(x, edge_index, W_l0, b_l0, W_r0, W_l1, b_l1, W_r1, W_l2, b_l2,
           W_r2):
    src = edge_index[0].astype(jnp.int32)
    dst = edge_index[1].astype(jnp.int32)
    npad_e = EPAD - E
    ar = jnp.arange(npad_e, dtype=jnp.int32)
    src2d = jnp.concatenate([src, ar % N]).reshape(EPAD // B, B)
    dst2d = jnp.concatenate([dst, N + ar % (NPAD - N)]).reshape(EPAD // B, B)

    segsum_cnt = _make_segsum(D, True)
    segsum_d = _make_segsum(D, False)
    segsum_f2 = segsum_d

    parts0, cntp = segsum_cnt(x, src2d, dst2d)
    h_lo, h_hi, invc = pl.pallas_call(
        _tc0_body,
        grid=(GRID,),
        in_specs=[
            pl.BlockSpec((2, RB, D), lambda i: (0, i, 0)),
            pl.BlockSpec((2, RB), lambda i: (0, i)),
            pl.BlockSpec((RB, D), lambda i: (i, 0)),
            pl.BlockSpec((D, H), lambda i: (0, 0)),
            pl.BlockSpec((1, H), lambda i: (0, 0)),
            pl.BlockSpec((D, H), lambda i: (0, 0)),
        ],
        out_specs=[
            pl.BlockSpec((RB, D), lambda i: (i, 0)),
            pl.BlockSpec((RB, D), lambda i: (i, 0)),
            pl.BlockSpec((RB,), lambda i: (i,)),
        ],
        out_shape=[
            jax.ShapeDtypeStruct((N, D), jnp.float32),
            jax.ShapeDtypeStruct((N, D), jnp.float32),
            jax.ShapeDtypeStruct((NPAD,), jnp.float32),
        ],
    )(parts0, cntp, x, W_r0, b_l0.reshape(1, H), W_l0)

    parts1_lo = segsum_d(h_lo, src2d, dst2d)[0]
    parts1_hi = segsum_d(h_hi, src2d, dst2d)[0]
    wl2p = jnp.concatenate([W_l2, jnp.zeros((H, F2 - C), jnp.float32)], axis=1)
    h2, t = pl.pallas_call(
        _tc1_body,
        grid=(GRID,),
        in_specs=[
            pl.BlockSpec((2, RB, D), lambda i: (0, i, 0)),
            pl.BlockSpec((2, RB, D), lambda i: (0, i, 0)),
            pl.BlockSpec((RB,), lambda i: (i,)),
            pl.BlockSpec((RB, D), lambda i: (i, 0)),
            pl.BlockSpec((RB, D), lambda i: (i, 0)),
            pl.BlockSpec((D, H), lambda i: (0, 0)),
            pl.BlockSpec((D, H), lambda i: (0, 0)),
            pl.BlockSpec((1, H), lambda i: (0, 0)),
            pl.BlockSpec((D, H), lambda i: (0, 0)),
            pl.BlockSpec((D, H), lambda i: (0, 0)),
            pl.BlockSpec((H, F2), lambda i: (0, 0)),
        ],
        out_specs=[
            pl.BlockSpec((RB, H), lambda i: (i, 0)),
            pl.BlockSpec((RB, F2), lambda i: (i, 0)),
        ],
        out_shape=[
            jax.ShapeDtypeStruct((N, H), jnp.float32),
            jax.ShapeDtypeStruct((N, F2), jnp.float32),
        ],
    )(parts1_lo, parts1_hi, invc, h_lo, h_hi, W_r1[:D], W_r1[D:],
      b_l1.reshape(1, H), W_l1[:D], W_l1[D:], wl2p)

    parts2 = segsum_f2(t, src2d, dst2d)[0]
    wr2p = jnp.concatenate([W_r2, jnp.zeros((H, F2 - C), jnp.float32)], axis=1)
    b2p = jnp.concatenate([b_l2, jnp.zeros((F2 - C,), jnp.float32)])
    out48 = pl.pallas_call(
        _tc2_body,
        grid=(GRID,),
        in_specs=[
            pl.BlockSpec((2, RB, F2), lambda i: (0, i, 0)),
            pl.BlockSpec((RB,), lambda i: (i,)),
            pl.BlockSpec((RB, H), lambda i: (i, 0)),
            pl.BlockSpec((H, F2), lambda i: (0, 0)),
            pl.BlockSpec((1, F2), lambda i: (0, 0)),
        ],
        out_specs=pl.BlockSpec((RB, F2), lambda i: (i, 0)),
        out_shape=jax.ShapeDtypeStruct((N, F2), jnp.float32),
    )(parts2, invc, h2, wr2p, b2p.reshape(1, F2))

    return out48[:, :C]
